# baseline (speedup 1.0000x reference)
import jax
import jax.numpy as jnp
from jax import lax
from jax.experimental import pallas as pl
from jax.experimental.pallas import tpu as pltpu
from jax.experimental.pallas import tpu_sc as plsc

_N = 100000
_E = 3200000
_H = 16

_NC = 2
_NS = 16
_NW = _NC * _NS
_GRP = 128
_GPD = 12
_CHUNK = _GRP * _GPD
_KITER = 66
_EPT = _CHUNK * _KITER
_EPAD = _EPT * _NW
_NACC = 100096
_ZPT = _NACC // _NS

_mesh = plsc.VectorSubcoreMesh(core_axis_name="c", subcore_axis_name="s")
_sc_params = pltpu.CompilerParams(use_tc_tiling_on_sc=False)


def _zero_acc(rows, acc, s):
    @pl.loop(0, _CHUNK)
    def _(i):
        rows[i, :] = jnp.zeros((16,), jnp.float32)

    base = s * _ZPT
    nfull = _ZPT // _CHUNK
    for z in range(nfull):
        pltpu.sync_copy(rows, acc.at[pl.ds(base + z * _CHUNK, _CHUNK)])
    rem = _ZPT - nfull * _CHUNK
    if rem:
        pltpu.sync_copy(rows.at[pl.ds(0, rem)],
                        acc.at[pl.ds(base + nfull * _CHUNK, rem)])


def _dump_acc(acc, p0_hbm, p1_hbm, c, s):
    base = s * _ZPT

    @pl.when(c == 0)
    def _():
        pltpu.sync_copy(acc.at[pl.ds(base, _ZPT)], p0_hbm.at[pl.ds(base, _ZPT)])

    @pl.when(c == 1)
    def _():
        pltpu.sync_copy(acc.at[pl.ds(base, _ZPT)], p1_hbm.at[pl.ds(base, _ZPT)])


def _gather_scatter_body(table_hbm, idx_hbm, p0_hbm, p1_hbm,
                         idxa, idxb, rows, acc, gsem, ssem, isem):
    c = lax.axis_index("c")
    s = lax.axis_index("s")
    wid = c * _NS + s
    g0 = wid * _KITER

    _zero_acc(rows, acc, s)
    plsc.subcore_barrier()

    def gslice(j):
        return rows.at[pl.ds(j * _GRP, _GRP)]

    def run_chunk(k, idx, nidx, prefetch):
        @pl.when(prefetch)
        def _():
            pltpu.async_copy(idx_hbm.at[g0 + k + 1], nidx, isem)

        for j in range(_GPD):
            pltpu.async_copy(
                table_hbm.at[idx.at[0].at[pl.ds(j * _GRP, _GRP)]],
                gslice(j), gsem)
        for j in range(_GPD):
            pltpu.make_async_copy(
                table_hbm.at[idx.at[0].at[pl.ds(j * _GRP, _GRP)]],
                gslice(j), gsem).wait()
            pltpu.async_copy(
                gslice(j), acc.at[idx.at[1].at[pl.ds(j * _GRP, _GRP)]],
                ssem, add=True)
        for j in range(_GPD):
            pltpu.make_async_copy(
                gslice(j), acc.at[idx.at[1].at[pl.ds(j * _GRP, _GRP)]],
                ssem).wait()

        @pl.when(prefetch)
        def _():
            pltpu.make_async_copy(idx_hbm.at[g0 + k + 1], nidx, isem).wait()

    pltpu.sync_copy(idx_hbm.at[g0], idxa)

    @pl.loop(0, _KITER // 2)
    def _(m):
        run_chunk(2 * m, idxa, idxb, True)
        run_chunk(2 * m + 1, idxb, idxa, m < _KITER // 2 - 1)

    plsc.subcore_barrier()
    _dump_acc(acc, p0_hbm, p1_hbm, c, s)


def _degree_body(idx_hbm, p0_hbm, p1_hbm, dstv, ones, acc, ssem0):
    c = lax.axis_index("c")
    s = lax.axis_index("s")
    wid = c * _NS + s
    g0 = wid * _KITER

    @pl.loop(0, _CHUNK // 16)
    def _(i):
        ones[pl.ds(i * 16, 16)] = jnp.zeros((16,), jnp.float32)

    base = s * _ZPT
    nfull = _ZPT // _CHUNK
    for z in range(nfull):
        pltpu.sync_copy(ones, acc.at[pl.ds(base + z * _CHUNK, _CHUNK)])
    rem = _ZPT - nfull * _CHUNK
    if rem:
        pltpu.sync_copy(ones.at[pl.ds(0, rem)],
                        acc.at[pl.ds(base + nfull * _CHUNK, rem)])

    @pl.loop(0, _GRP // 16)
    def _(i):
        ones[pl.ds(i * 16, 16)] = jnp.ones((16,), jnp.float32)

    plsc.subcore_barrier()

    src1 = ones.at[pl.ds(0, _GRP)]

    @pl.loop(0, _KITER)
    def _(k):
        pltpu.sync_copy(idx_hbm.at[g0 + k].at[1], dstv)
        for j in range(_GPD):
            pltpu.async_copy(
                src1, acc.at[dstv.at[pl.ds(j * _GRP, _GRP)]],
                ssem0, add=True)
        for j in range(_GPD):
            pltpu.make_async_copy(
                src1, acc.at[dstv.at[pl.ds(j * _GRP, _GRP)]],
                ssem0).wait()

    plsc.subcore_barrier()

    @pl.when(c == 0)
    def _():
        pltpu.sync_copy(acc.at[pl.ds(base, _ZPT)], p0_hbm.at[pl.ds(base, _ZPT)])

    @pl.when(c == 1)
    def _():
        pltpu.sync_copy(acc.at[pl.ds(base, _ZPT)], p1_hbm.at[pl.ds(base, _ZPT)])


@jax.jit
def _sc_edge_pass(table, idx4):
    out = jax.ShapeDtypeStruct((_NACC, _H), jnp.float32)
    f = pl.kernel(
        _gather_scatter_body,
        out_type=[out, out],
        mesh=_mesh,
        scratch_types=[
            pltpu.VMEM((2, _CHUNK), jnp.int32),
            pltpu.VMEM((2, _CHUNK), jnp.int32),
            pltpu.VMEM((_CHUNK, _H), jnp.float32),
            pltpu.VMEM_SHARED((_NACC, _H), jnp.float32),
            pltpu.SemaphoreType.DMA,
            pltpu.SemaphoreType.DMA,
            pltpu.SemaphoreType.DMA,
        ],
        compiler_params=_sc_params,
    )
    return f(table, idx4)


@jax.jit
def _sc_degree_pass(idx4):
    out = jax.ShapeDtypeStruct((_NACC,), jnp.float32)
    f = pl.kernel(
        _degree_body,
        out_type=[out, out],
        mesh=_mesh,
        scratch_types=[
            pltpu.VMEM((_CHUNK,), jnp.int32),
            pltpu.VMEM((_CHUNK,), jnp.float32),
            pltpu.VMEM_SHARED((_NACC,), jnp.float32),
            pltpu.SemaphoreType.DMA,
        ],
        compiler_params=_sc_params,
    )
    return f(idx4)


_F = 128
_NRF = _NACC // 8
_RBF = 544
_NBF = _NRF // _RBF

_blk = lambda: pl.BlockSpec((_RBF, _F), lambda i: (i, 0))
_wblk = lambda: pl.BlockSpec((_F, _F), lambda i: (0, 0))
_bblk = lambda: pl.BlockSpec((1, _F), lambda i: (0, 0))
_o16 = jax.ShapeDtypeStruct((_NRF, _F), jnp.float32)


def _k_pre(pd0, pd1, x16, w1, e8, dinv_ref, t1_ref):
    deg8 = pd0[...] + pd1[...] + 1.0
    dinv8 = lax.rsqrt(deg8)
    dinv = jnp.dot(dinv8, e8[...], preferred_element_type=jnp.float32)
    dinv_ref[...] = dinv
    t1_ref[...] = dinv * jnp.dot(x16[...], w1[...],
                                 preferred_element_type=jnp.float32)


def _k_layer1(p0, p1, t1, dinv, w2, b1, h1_ref, t2_ref):
    d = dinv[...]
    h1 = jnp.maximum(d * (p0[...] + p1[...] + t1[...]) + b1[...], 0.0)
    h1_ref[...] = h1
    t2_ref[...] = d * jnp.dot(h1, w2[...], preferred_element_type=jnp.float32)


def _k_layer2(p0, p1, t2, dinv, w3, b2, t3_ref):
    d = dinv[...]
    h2 = d * (p0[...] + p1[...] + t2[...]) + b2[...]
    t3_ref[...] = d * jnp.dot(h2, w3[...], preferred_element_type=jnp.float32)


def _k_layer3(p0, p1, t3, dinv, h1, b3, t4_ref):
    d = dinv[...]
    h3 = jnp.maximum(d * (p0[...] + p1[...] + t3[...]) + b3[...] + h1[...], 0.0)
    t4_ref[...] = d * h3


def _k_layer4(p0, p1, t4, dinv, w4, b4, out_ref):
    z = dinv[...] * (p0[...] + p1[...] + t4[...])
    out_ref[...] = jnp.dot(z, w4[...],
                           preferred_element_type=jnp.float32) + b4[...]


def _tc_call(body, ins, in_specs, n_out):
    return pl.pallas_call(
        body,
        grid=(_NBF,),
        in_specs=in_specs,
        out_specs=[_blk() for _ in range(n_out)],
        out_shape=[_o16] * n_out,
    )(*ins)


def _bd(w):
    return jnp.kron(jnp.eye(8, dtype=jnp.float32), w)


def _btile(b):
    return jnp.tile(b, 8).reshape(1, _F)


def kernel(x, edge_index, W1, b1, W2, b2, W3, b3, W4, b4):
    src = edge_index[0]
    dst = edge_index[1]
    pad = _EPAD - _E
    src3 = jnp.concatenate(
        [src, jnp.zeros((pad,), jnp.int32)]).reshape(-1, _CHUNK)
    dst3 = jnp.concatenate(
        [dst, jnp.full((pad,), _N, jnp.int32)]).reshape(-1, _CHUNK)
    idx4 = jnp.stack([src3, dst3], axis=1)

    x16f = jnp.pad(x, ((0, _NACC - _N), (0, _H - x.shape[1]))).reshape(_NRF, _F)
    w1bd = _bd(jnp.pad(W1, ((0, _H - W1.shape[0]), (0, 0))))
    w2bd = _bd(W2)
    w3bd = _bd(W3)
    w4bd = _bd(jnp.pad(W4, ((0, 0), (0, _H - W4.shape[1]))))
    b4t = _btile(jnp.pad(b4, ((0, _H - b4.shape[0]),)))
    b1t = _btile(b1)
    b2t = _btile(b2)
    b3t = _btile(b3)

    def flat(p):
        return p.reshape(_NRF, _F)

    pd0, pd1 = _sc_degree_pass(idx4)
    e8 = jnp.kron(jnp.eye(8, dtype=jnp.float32), jnp.ones((1, _H), jnp.float32))
    d8blk = pl.BlockSpec((_RBF, 8), lambda i: (i, 0))
    e8blk = pl.BlockSpec((8, _F), lambda i: (0, 0))

    dinv, t1 = _tc_call(
        _k_pre, (pd0.reshape(_NRF, 8), pd1.reshape(_NRF, 8), x16f, w1bd, e8),
        [d8blk, d8blk, _blk(), _wblk(), e8blk], 2)

    p0, p1 = _sc_edge_pass(t1.reshape(_NACC, _H), idx4)
    h1, t2 = _tc_call(
        _k_layer1, (flat(p0), flat(p1), t1, dinv, w2bd, b1t),
        [_blk(), _blk(), _blk(), _blk(), _wblk(), _bblk()], 2)

    p0, p1 = _sc_edge_pass(t2.reshape(_NACC, _H), idx4)
    (t3,) = _tc_call(
        _k_layer2, (flat(p0), flat(p1), t2, dinv, w3bd, b2t),
        [_blk(), _blk(), _blk(), _blk(), _wblk(), _bblk()], 1)

    p0, p1 = _sc_edge_pass(t3.reshape(_NACC, _H), idx4)
    (t4,) = _tc_call(
        _k_layer3, (flat(p0), flat(p1), t3, dinv, h1, b3t),
        [_blk(), _blk(), _blk(), _blk(), _blk(), _bblk()], 1)

    p0, p1 = _sc_edge_pass(t4.reshape(_NACC, _H), idx4)
    (out16,) = _tc_call(
        _k_layer4, (flat(p0), flat(p1), t4, dinv, w4bd, b4t),
        [_blk(), _blk(), _blk(), _blk(), _wblk(), _bblk()], 1)

    return out16.reshape(_NACC, _H)[:_N, : W4.shape[1]].astype(jnp.float32)


# speedup vs baseline: 1.4503x; 1.4503x over previous
import jax
import jax.numpy as jnp
from jax import lax
from jax.experimental import pallas as pl
from jax.experimental.pallas import tpu as pltpu
from jax.experimental.pallas import tpu_sc as plsc

_N = 100000
_E = 3200000
_H = 16

_NC = 2
_NS = 16
_NW = _NC * _NS
_GRP = 128
_GPD = 8
_CHUNK = _GRP * _GPD
_KITER = 98
_EPT = _CHUNK * _KITER
_EPAD = _EPT * _NW
_NACC = 100096
_ZPT = _NACC // _NS

_mesh = plsc.VectorSubcoreMesh(core_axis_name="c", subcore_axis_name="s")
_sc_params = pltpu.CompilerParams(use_tc_tiling_on_sc=False)


def _zero_acc(rows, acc, s):
    @pl.loop(0, _CHUNK)
    def _(i):
        rows[i, :] = jnp.zeros((16,), jnp.float32)

    base = s * _ZPT
    nfull = _ZPT // _CHUNK
    for z in range(nfull):
        pltpu.sync_copy(rows, acc.at[pl.ds(base + z * _CHUNK, _CHUNK)])
    rem = _ZPT - nfull * _CHUNK
    if rem:
        pltpu.sync_copy(rows.at[pl.ds(0, rem)],
                        acc.at[pl.ds(base + nfull * _CHUNK, rem)])


def _dump_acc(acc, p0_hbm, p1_hbm, c, s):
    base = s * _ZPT

    @pl.when(c == 0)
    def _():
        pltpu.sync_copy(acc.at[pl.ds(base, _ZPT)], p0_hbm.at[pl.ds(base, _ZPT)])

    @pl.when(c == 1)
    def _():
        pltpu.sync_copy(acc.at[pl.ds(base, _ZPT)], p1_hbm.at[pl.ds(base, _ZPT)])


def _gather_scatter_body(table_hbm, idx_hbm, p0_hbm, p1_hbm,
                         idxa, idxb, rows, acc, gsem, ssem, isem):
    c = lax.axis_index("c")
    s = lax.axis_index("s")
    wid = c * _NS + s
    g0 = wid * _KITER

    _zero_acc(rows, acc, s)
    plsc.subcore_barrier()

    def gslice(j):
        return rows.at[pl.ds(j * _GRP, _GRP)]

    def run_chunk(k, idx, nidx, prefetch):
        @pl.when(prefetch)
        def _():
            pltpu.async_copy(idx_hbm.at[g0 + k + 1], nidx, isem)

        for j in range(_GPD):
            pltpu.async_copy(
                table_hbm.at[idx.at[0].at[pl.ds(j * _GRP, _GRP)]],
                gslice(j), gsem)
        for j in range(_GPD):
            pltpu.make_async_copy(
                table_hbm.at[idx.at[0].at[pl.ds(j * _GRP, _GRP)]],
                gslice(j), gsem).wait()
            pltpu.async_copy(
                gslice(j), acc.at[idx.at[1].at[pl.ds(j * _GRP, _GRP)]],
                ssem, add=True)
        for j in range(_GPD):
            pltpu.make_async_copy(
                gslice(j), acc.at[idx.at[1].at[pl.ds(j * _GRP, _GRP)]],
                ssem).wait()

        @pl.when(prefetch)
        def _():
            pltpu.make_async_copy(idx_hbm.at[g0 + k + 1], nidx, isem).wait()

    pltpu.sync_copy(idx_hbm.at[g0], idxa)

    @pl.loop(0, _KITER // 2)
    def _(m):
        run_chunk(2 * m, idxa, idxb, True)
        run_chunk(2 * m + 1, idxb, idxa, m < _KITER // 2 - 1)

    plsc.subcore_barrier()
    _dump_acc(acc, p0_hbm, p1_hbm, c, s)


def _degree_body(idx_hbm, p0_hbm, p1_hbm, dstv, ones, acc, ssem0):
    c = lax.axis_index("c")
    s = lax.axis_index("s")
    wid = c * _NS + s
    g0 = wid * _KITER

    @pl.loop(0, _CHUNK // 16)
    def _(i):
        ones[pl.ds(i * 16, 16)] = jnp.zeros((16,), jnp.float32)

    base = s * _ZPT
    nfull = _ZPT // _CHUNK
    for z in range(nfull):
        pltpu.sync_copy(ones, acc.at[pl.ds(base + z * _CHUNK, _CHUNK)])
    rem = _ZPT - nfull * _CHUNK
    if rem:
        pltpu.sync_copy(ones.at[pl.ds(0, rem)],
                        acc.at[pl.ds(base + nfull * _CHUNK, rem)])

    @pl.loop(0, _GRP // 16)
    def _(i):
        ones[pl.ds(i * 16, 16)] = jnp.ones((16,), jnp.float32)

    plsc.subcore_barrier()

    src1 = ones.at[pl.ds(0, _GRP)]

    @pl.loop(0, _KITER)
    def _(k):
        pltpu.sync_copy(idx_hbm.at[g0 + k].at[1], dstv)
        for j in range(_GPD):
            pltpu.async_copy(
                src1, acc.at[dstv.at[pl.ds(j * _GRP, _GRP)]],
                ssem0, add=True)
        for j in range(_GPD):
            pltpu.make_async_copy(
                src1, acc.at[dstv.at[pl.ds(j * _GRP, _GRP)]],
                ssem0).wait()

    plsc.subcore_barrier()

    @pl.when(c == 0)
    def _():
        pltpu.sync_copy(acc.at[pl.ds(base, _ZPT)], p0_hbm.at[pl.ds(base, _ZPT)])

    @pl.when(c == 1)
    def _():
        pltpu.sync_copy(acc.at[pl.ds(base, _ZPT)], p1_hbm.at[pl.ds(base, _ZPT)])


@jax.jit
def _sc_edge_pass(table, idx4):
    out = jax.ShapeDtypeStruct((_NACC, _H), jnp.float32)
    f = pl.kernel(
        _gather_scatter_body,
        out_type=[out, out],
        mesh=_mesh,
        scratch_types=[
            pltpu.VMEM((2, _CHUNK), jnp.int32),
            pltpu.VMEM((2, _CHUNK), jnp.int32),
            pltpu.VMEM((_CHUNK, _H), jnp.float32),
            pltpu.VMEM_SHARED((_NACC, _H), jnp.float32),
            pltpu.SemaphoreType.DMA,
            pltpu.SemaphoreType.DMA,
            pltpu.SemaphoreType.DMA,
        ],
        compiler_params=_sc_params,
    )
    return f(table, idx4)


@jax.jit
def _sc_degree_pass(idx4):
    out = jax.ShapeDtypeStruct((_NACC,), jnp.float32)
    f = pl.kernel(
        _degree_body,
        out_type=[out, out],
        mesh=_mesh,
        scratch_types=[
            pltpu.VMEM((_CHUNK,), jnp.int32),
            pltpu.VMEM((_CHUNK,), jnp.float32),
            pltpu.VMEM_SHARED((_NACC,), jnp.float32),
            pltpu.SemaphoreType.DMA,
        ],
        compiler_params=_sc_params,
    )
    return f(idx4)


_F = 128
_NRF = _NACC // 8
_RBF = 544
_NBF = _NRF // _RBF

_blk = lambda: pl.BlockSpec((_RBF, _F), lambda i: (i, 0))
_wblk = lambda: pl.BlockSpec((_F, _F), lambda i: (0, 0))
_bblk = lambda: pl.BlockSpec((1, _F), lambda i: (0, 0))
_o16 = jax.ShapeDtypeStruct((_NRF, _F), jnp.float32)


def _k_pre(pd0, pd1, x16, w1, e8, dinv_ref, t1_ref):
    deg8 = pd0[...] + pd1[...] + 1.0
    dinv8 = lax.rsqrt(deg8)
    dinv = jnp.dot(dinv8, e8[...], preferred_element_type=jnp.float32)
    dinv_ref[...] = dinv
    t1_ref[...] = dinv * jnp.dot(x16[...], w1[...],
                                 preferred_element_type=jnp.float32)


def _k_layer1(p0, p1, t1, dinv, w2, b1, h1_ref, t2_ref):
    d = dinv[...]
    h1 = jnp.maximum(d * (p0[...] + p1[...] + t1[...]) + b1[...], 0.0)
    h1_ref[...] = h1
    t2_ref[...] = d * jnp.dot(h1, w2[...], preferred_element_type=jnp.float32)


def _k_layer2(p0, p1, t2, dinv, w3, b2, t3_ref):
    d = dinv[...]
    h2 = d * (p0[...] + p1[...] + t2[...]) + b2[...]
    t3_ref[...] = d * jnp.dot(h2, w3[...], preferred_element_type=jnp.float32)


def _k_layer3(p0, p1, t3, dinv, h1, b3, t4_ref):
    d = dinv[...]
    h3 = jnp.maximum(d * (p0[...] + p1[...] + t3[...]) + b3[...] + h1[...], 0.0)
    t4_ref[...] = d * h3


def _k_layer4(p0, p1, t4, dinv, w4, b4, out_ref):
    z = dinv[...] * (p0[...] + p1[...] + t4[...])
    out_ref[...] = jnp.dot(z, w4[...],
                           preferred_element_type=jnp.float32) + b4[...]


def _tc_call(body, ins, in_specs, n_out):
    return pl.pallas_call(
        body,
        grid=(_NBF,),
        in_specs=in_specs,
        out_specs=[_blk() for _ in range(n_out)],
        out_shape=[_o16] * n_out,
    )(*ins)


def _bd(w):
    return jnp.kron(jnp.eye(8, dtype=jnp.float32), w)


def _btile(b):
    return jnp.tile(b, 8).reshape(1, _F)


def kernel(x, edge_index, W1, b1, W2, b2, W3, b3, W4, b4):
    src = edge_index[0]
    dst = edge_index[1]
    pad = _EPAD - _E
    src3 = jnp.concatenate(
        [src, jnp.zeros((pad,), jnp.int32)]).reshape(-1, _CHUNK)
    dst3 = jnp.concatenate(
        [dst, jnp.full((pad,), _N, jnp.int32)]).reshape(-1, _CHUNK)
    idx4 = jnp.stack([src3, dst3], axis=1)

    x16f = jnp.pad(x, ((0, _NACC - _N), (0, _H - x.shape[1]))).reshape(_NRF, _F)
    w1bd = _bd(jnp.pad(W1, ((0, _H - W1.shape[0]), (0, 0))))
    w2bd = _bd(W2)
    w3bd = _bd(W3)
    w4bd = _bd(jnp.pad(W4, ((0, 0), (0, _H - W4.shape[1]))))
    b4t = _btile(jnp.pad(b4, ((0, _H - b4.shape[0]),)))
    b1t = _btile(b1)
    b2t = _btile(b2)
    b3t = _btile(b3)

    def flat(p):
        return p.reshape(_NRF, _F)

    pd0, pd1 = _sc_degree_pass(idx4)
    e8 = jnp.kron(jnp.eye(8, dtype=jnp.float32), jnp.ones((1, _H), jnp.float32))
    d8blk = pl.BlockSpec((_RBF, 8), lambda i: (i, 0))
    e8blk = pl.BlockSpec((8, _F), lambda i: (0, 0))

    dinv, t1 = _tc_call(
        _k_pre, (pd0.reshape(_NRF, 8), pd1.reshape(_NRF, 8), x16f, w1bd, e8),
        [d8blk, d8blk, _blk(), _wblk(), e8blk], 2)

    p0, p1 = _sc_edge_pass(t1.reshape(_NACC, _H), idx4)
    h1, t2 = _tc_call(
        _k_layer1, (flat(p0), flat(p1), t1, dinv, w2bd, b1t),
        [_blk(), _blk(), _blk(), _blk(), _wblk(), _bblk()], 2)

    p0, p1 = _sc_edge_pass(t2.reshape(_NACC, _H), idx4)
    (t3,) = _tc_call(
        _k_layer2, (flat(p0), flat(p1), t2, dinv, w3bd, b2t),
        [_blk(), _blk(), _blk(), _blk(), _wblk(), _bblk()], 1)

    p0, p1 = _sc_edge_pass(t3.reshape(_NACC, _H), idx4)
    (t4,) = _tc_call(
        _k_layer3, (flat(p0), flat(p1), t3, dinv, h1, b3t),
        [_blk(), _blk(), _blk(), _blk(), _blk(), _bblk()], 1)

    p0, p1 = _sc_edge_pass(t4.reshape(_NACC, _H), idx4)
    (out16,) = _tc_call(
        _k_layer4, (flat(p0), flat(p1), t4, dinv, w4bd, b4t),
        [_blk(), _blk(), _blk(), _blk(), _wblk(), _bblk()], 1)

    return out16.reshape(_NACC, _H)[:_N, : W4.shape[1]].astype(jnp.float32)
